# Initial kernel scaffold; baseline (speedup 1.0000x reference)
import jax
import jax.numpy as jnp
from jax.experimental import pallas as pl


def kernel(indices, table):
    raise NotImplementedError("write your pallas kernel here")


# speedup vs baseline: 1.1124x; 1.1124x over previous
import functools

import jax
import jax.numpy as jnp
from jax import lax
from jax.experimental import pallas as pl
from jax.experimental.pallas import tpu as pltpu
from jax.experimental.pallas import tpu_sc as plsc

NC = 2
NS = 16
NW = NC * NS
K = 128
NBUF = 8


@functools.lru_cache(maxsize=None)
def _make_emb(n_chunks_w: int, n_rows: int, d: int):
    n_groups = n_chunks_w // NBUF
    n_pairs = n_groups // 2
    mesh = plsc.VectorSubcoreMesh(core_axis_name="c", subcore_axis_name="s")

    @functools.partial(
        pl.kernel,
        mesh=mesh,
        compiler_params=pltpu.CompilerParams(use_tc_tiling_on_sc=False),
        out_type=jax.ShapeDtypeStruct((NW * n_chunks_w * K, d), jnp.float32),
        scratch_types=[
            pltpu.VMEM((n_chunks_w, K), jnp.int32),
            pltpu.VMEM((2 * NBUF, K, d), jnp.float32),
            pltpu.SemaphoreType.DMA,
            pltpu.SemaphoreType.DMA,
            pltpu.SemaphoreType.DMA,
            pltpu.SemaphoreType.DMA,
        ],
    )
    def emb(idx_hbm, table_hbm, out_hbm, idx_v, rows_v, gsa, gsb, ssa, ssb):
        wid = lax.axis_index("s") * NC + lax.axis_index("c")
        base = wid * (n_chunks_w * K)
        pltpu.sync_copy(idx_hbm.at[wid], idx_v)

        def fire_gathers(g, set_id, sem):
            for b in range(NBUF):
                chunk = g * NBUF + b
                pltpu.make_async_copy(
                    table_hbm.at[idx_v.at[chunk]],
                    rows_v.at[set_id * NBUF + b], sem).start()

        def drain_gathers(set_id, sem):
            for b in range(NBUF):
                pltpu.make_async_copy(
                    table_hbm.at[pl.ds(0, K)],
                    rows_v.at[set_id * NBUF + b], sem).wait()

        def fire_stores(g, set_id, sem):
            for b in range(NBUF):
                chunk = g * NBUF + b
                pltpu.make_async_copy(
                    rows_v.at[set_id * NBUF + b],
                    out_hbm.at[pl.ds(base + chunk * K, K)], sem).start()

        def drain_stores(set_id, sem):
            for b in range(NBUF):
                pltpu.make_async_copy(
                    rows_v.at[set_id * NBUF + b],
                    out_hbm.at[pl.ds(0, K)], sem).wait()

        fire_gathers(0, 0, gsa)

        def pair(i, carry):
            g = 2 * i
            fire_gathers(g + 1, 1, gsb)
            drain_gathers(0, gsa)
            fire_stores(g, 0, ssa)
            drain_stores(0, ssa)
            fire_gathers(g + 2, 0, gsa)
            drain_gathers(1, gsb)
            fire_stores(g + 1, 1, ssb)
            drain_stores(1, ssb)
            return carry

        lax.fori_loop(0, n_pairs - 1, pair, 0)

        g_last = 2 * (n_pairs - 1)
        fire_gathers(g_last + 1, 1, gsb)
        drain_gathers(0, gsa)
        fire_stores(g_last, 0, ssa)
        drain_stores(0, ssa)
        drain_gathers(1, gsb)
        fire_stores(g_last + 1, 1, ssb)
        drain_stores(1, ssb)

    return emb


def kernel(indices, table):
    d = table.shape[1]
    idx = indices.astype(jnp.int32).reshape(-1)
    b_total = idx.shape[0]
    assert b_total % (NW * K * 2 * NBUF) == 0, b_total
    n_chunks_w = b_total // (NW * K)
    idx3 = idx.reshape(NW, n_chunks_w, K)
    out = _make_emb(n_chunks_w, table.shape[0], d)(idx3, table)
    return out.reshape(indices.shape + (d,))


# speedup vs baseline: 3.4676x; 3.1173x over previous
import functools

import jax
import jax.numpy as jnp
from jax import lax
from jax.experimental import pallas as pl
from jax.experimental.pallas import tpu as pltpu
from jax.experimental.pallas import tpu_sc as plsc

NC = 2
NS = 16
NW = NC * NS
K = 128
NSLOT = 4


@functools.lru_cache(maxsize=None)
def _make_emb(n_t: int, n_bt: int, n_rows: int, d: int):
    n_chunks = n_t * n_bt
    cpw = n_chunks // NW
    n_er = d // 8
    t_stride = n_er * n_bt * 8 * K
    r_stride = n_bt * 8 * K
    out_words = n_t * t_stride
    mesh = plsc.VectorSubcoreMesh(core_axis_name="c", subcore_axis_name="s")

    @functools.partial(
        pl.kernel,
        mesh=mesh,
        compiler_params=pltpu.CompilerParams(
            use_tc_tiling_on_sc=False, needs_layout_passes=False),
        out_type=jax.ShapeDtypeStruct((out_words,), jnp.float32),
        scratch_types=[
            pltpu.VMEM((cpw, K), jnp.int32),
            pltpu.VMEM((NSLOT, K, d), jnp.float32),
            pltpu.VMEM((NSLOT, n_er, 8 * K), jnp.float32),
        ]
        + [pltpu.SemaphoreType.DMA] * (2 * NSLOT),
    )
    def emb(idx_hbm, table_hbm, out_hbm, idx_v, rows_v, tbuf_v, *sems):
        gsems, ssems = sems[:NSLOT], sems[NSLOT:]
        wid = lax.axis_index("s") * NC + lax.axis_index("c")
        m0 = wid * cpw
        pltpu.sync_copy(idx_hbm.at[wid], idx_v)

        row_iota = [lax.iota(jnp.int32, 16) + 16 * jb for jb in range(8)]
        col_e = [jnp.full((16,), e, jnp.int32) for e in range(d)]

        def fire_gather(local, s):
            pltpu.make_async_copy(
                table_hbm.at[idx_v.at[local]], rows_v.at[s], gsems[s]).start()

        for s in range(NSLOT):
            fire_gather(s, s)

        def body(it, carry):
            i = it * NSLOT
            for s in range(NSLOT):
                local = i + s
                m = m0 + local
                t = m // n_bt
                c = m - t * n_bt
                out_base = t * t_stride + c * (8 * K)
                pltpu.make_async_copy(
                    table_hbm.at[pl.ds(0, K)], rows_v.at[s], gsems[s]).wait()

                @pl.when(it > 0)
                def _drain():
                    for r in range(n_er):
                        pltpu.make_async_copy(
                            tbuf_v.at[s, r], out_hbm.at[pl.ds(0, 8 * K)],
                            ssems[s]).wait()

                for e in range(d):
                    r, iy = e // 8, e % 8
                    for jb in range(8):
                        v = plsc.load_gather(
                            rows_v.at[s], [row_iota[jb], col_e[e]])
                        tbuf_v[s, r, pl.ds(iy * K + jb * 16, 16)] = v
                for r in range(n_er):
                    pltpu.make_async_copy(
                        tbuf_v.at[s, r],
                        out_hbm.at[pl.ds(out_base + r * r_stride, 8 * K)],
                        ssems[s]).start()

                @pl.when(local + NSLOT < cpw)
                def _refire():
                    fire_gather(local + NSLOT, s)
            return carry

        lax.fori_loop(0, cpw // NSLOT, body, 0)
        for s in range(NSLOT):
            for r in range(n_er):
                pltpu.make_async_copy(
                    tbuf_v.at[s, r], out_hbm.at[pl.ds(0, 8 * K)],
                    ssems[s]).wait()

    return emb


def kernel(indices, table):
    n_b, n_t = indices.shape
    n_rows, d = table.shape
    assert n_b % K == 0 and d % 8 == 0
    n_bt = n_b // K
    assert (n_t * n_bt) % (NW * NSLOT) == 0
    idx_t = indices.astype(jnp.int32).T.reshape(-1)
    idx3 = idx_t.reshape(NW, (n_t * n_bt) // NW, K)
    out1 = _make_emb(n_t, n_bt, n_rows, d)(idx3, table)
    out5 = out1.reshape(n_t, d // 8, n_bt, 8, K)
    return out5.transpose(2, 4, 0, 1, 3).reshape(n_b, n_t, d)


# speedup vs baseline: 6.3616x; 1.8346x over previous
import functools

import jax
import jax.numpy as jnp
from jax import lax
from jax.experimental import pallas as pl
from jax.experimental.pallas import tpu as pltpu
from jax.experimental.pallas import tpu_sc as plsc

NC = 2
NS = 16
NW = NC * NS
K = 128
NSLOT = 4


@functools.lru_cache(maxsize=None)
def _make_emb(n_t: int, n_bt: int, n_rows: int, d: int):
    n_chunks = n_t * n_bt
    cpw = n_chunks // NW
    n_er = d // 8
    t_stride = n_er * n_bt * 8 * K
    r_stride = n_bt * 8 * K
    out_words = n_t * t_stride
    mesh = plsc.VectorSubcoreMesh(core_axis_name="c", subcore_axis_name="s")

    @functools.partial(
        pl.kernel,
        mesh=mesh,
        compiler_params=pltpu.CompilerParams(
            use_tc_tiling_on_sc=False, needs_layout_passes=False),
        out_type=jax.ShapeDtypeStruct((n_t, n_er, n_bt, 8, K), jnp.float32),
        scratch_types=[
            pltpu.VMEM((cpw, K), jnp.int32),
            pltpu.VMEM((NSLOT, K, d), jnp.float32),
            pltpu.VMEM((NSLOT, n_er, 8, K + 1), jnp.float32),
        ]
        + [pltpu.SemaphoreType.DMA] * (2 * NSLOT),
    )
    def emb(idx_hbm, table_hbm, out_hbm, idx_v, rows_v, tbuf_v, *sems):
        gsems, ssems = sems[:NSLOT], sems[NSLOT:]
        wid = lax.axis_index("s") * NC + lax.axis_index("c")
        m0 = wid * cpw
        pltpu.sync_copy(idx_hbm.at[wid], idx_v)

        lane = lax.iota(jnp.int32, 16)
        i_vec = lane & 7
        r_vec = [(lane >> 3) + 2 * h for h in range(d // 16)]
        z16 = lane * 0

        def fire_gather(local, s):
            pltpu.make_async_copy(
                table_hbm.at[idx_v.at[local]], rows_v.at[s], gsems[s]).start()

        for s in range(NSLOT):
            fire_gather(s, s)

        def body(it, carry):
            i = it * NSLOT
            for s in range(NSLOT):
                local = i + s
                m = m0 + local
                t = m // n_bt
                c = m - t * n_bt
                pltpu.make_async_copy(
                    table_hbm.at[pl.ds(0, K)], rows_v.at[s], gsems[s]).wait()

                @pl.when(it > 0)
                def _drain():
                    for r in range(n_er):
                        pltpu.make_async_copy(
                            tbuf_v.at[s, r, :, pl.ds(0, K)],
                            out_hbm.at[0, r, 0], ssems[s]).wait()

                for j in range(K):
                    for h in range(d // 16):
                        x = rows_v[s, j, pl.ds(16 * h, 16)]
                        plsc.store_scatter(
                            tbuf_v.at[s], [r_vec[h], i_vec, z16 + j], x)
                for r in range(n_er):
                    pltpu.make_async_copy(
                        tbuf_v.at[s, r, :, pl.ds(0, K)],
                        out_hbm.at[t, r, c], ssems[s]).start()

                @pl.when(local + NSLOT < cpw)
                def _refire():
                    fire_gather(local + NSLOT, s)
            return carry

        lax.fori_loop(0, cpw // NSLOT, body, 0)
        for s in range(NSLOT):
            for r in range(n_er):
                pltpu.make_async_copy(
                    tbuf_v.at[s, r, :, pl.ds(0, K)],
                    out_hbm.at[0, r, 0], ssems[s]).wait()

    return emb


def kernel(indices, table):
    n_b, n_t = indices.shape
    n_rows, d = table.shape
    assert n_b % K == 0 and d % 8 == 0
    n_bt = n_b // K
    assert (n_t * n_bt) % (NW * NSLOT) == 0
    idx_t = indices.astype(jnp.int32).T.reshape(-1)
    idx3 = idx_t.reshape(NW, (n_t * n_bt) // NW, K)
    out5 = _make_emb(n_t, n_bt, n_rows, d)(idx3, table)
    return out5.transpose(2, 4, 0, 1, 3).reshape(n_b, n_t, d)


# speedup vs baseline: 10.2899x; 1.6175x over previous
import functools

import jax
import jax.numpy as jnp
from jax import lax
from jax.experimental import pallas as pl
from jax.experimental.pallas import tpu as pltpu
from jax.experimental.pallas import tpu_sc as plsc

NC = 2
NS = 16
NW = NC * NS
K = 128
NSLOT = 4


@functools.lru_cache(maxsize=None)
def _make_emb(n_t: int, n_bt: int, n_rows: int, d: int):
    n_chunks = n_t * n_bt
    cpw = n_chunks // NW
    n_er = d // 8
    t_stride = n_er * n_bt * 8 * K
    r_stride = n_bt * 8 * K
    out_words = n_t * t_stride
    mesh = plsc.VectorSubcoreMesh(core_axis_name="c", subcore_axis_name="s")

    @functools.partial(
        pl.kernel,
        mesh=mesh,
        compiler_params=pltpu.CompilerParams(
            use_tc_tiling_on_sc=False, needs_layout_passes=False,
            disable_bounds_checks=True),
        out_type=jax.ShapeDtypeStruct((n_t, n_er, n_bt, 8, K), jnp.float32),
        scratch_types=[
            pltpu.VMEM((cpw, K), jnp.int32),
            pltpu.VMEM((NSLOT, K, d), jnp.float32),
            pltpu.VMEM((NSLOT, n_er, 8, K + 1), jnp.float32),
        ]
        + [pltpu.SemaphoreType.DMA] * (2 * NSLOT),
    )
    def emb(idx_hbm, table_hbm, out_hbm, idx_v, rows_v, tbuf_v, *sems):
        gsems, ssems = sems[:NSLOT], sems[NSLOT:]
        wid = lax.axis_index("s") * NC + lax.axis_index("c")
        m0 = wid * cpw
        pltpu.sync_copy(idx_hbm.at[wid], idx_v)

        lane = lax.iota(jnp.int32, 16)
        i_vec = lane & 7
        r_vec = [(lane >> 3) + 2 * h for h in range(d // 16)]
        z16 = lane * 0

        def fire_gather(local, s):
            pltpu.make_async_copy(
                table_hbm.at[idx_v.at[local]], rows_v.at[s], gsems[s]).start()

        for s in range(NSLOT):
            fire_gather(s, s)

        def body(it, carry):
            i = it * NSLOT
            for s in range(NSLOT):
                local = i + s
                m = m0 + local
                t = m // n_bt
                c = m - t * n_bt
                pltpu.make_async_copy(
                    table_hbm.at[pl.ds(0, K)], rows_v.at[s], gsems[s]).wait()

                @pl.when(it > 0)
                def _drain():
                    for r in range(n_er):
                        pltpu.make_async_copy(
                            tbuf_v.at[s, r, :, pl.ds(0, K)],
                            out_hbm.at[0, r, 0], ssems[s]).wait()

                @plsc.parallel_loop(0, K, 1, unroll=8)
                def _transpose(j):
                    for h in range(d // 16):
                        x = rows_v[s, j, pl.ds(16 * h, 16)]
                        plsc.store_scatter(
                            tbuf_v.at[s], [r_vec[h], i_vec, z16 + j], x)
                for r in range(n_er):
                    pltpu.make_async_copy(
                        tbuf_v.at[s, r, :, pl.ds(0, K)],
                        out_hbm.at[t, r, c], ssems[s]).start()

                @pl.when(local + NSLOT < cpw)
                def _refire():
                    fire_gather(local + NSLOT, s)
            return carry

        lax.fori_loop(0, cpw // NSLOT, body, 0)
        for s in range(NSLOT):
            for r in range(n_er):
                pltpu.make_async_copy(
                    tbuf_v.at[s, r, :, pl.ds(0, K)],
                    out_hbm.at[0, r, 0], ssems[s]).wait()

    return emb


def kernel(indices, table):
    n_b, n_t = indices.shape
    n_rows, d = table.shape
    assert n_b % K == 0 and d % 8 == 0
    n_bt = n_b // K
    assert (n_t * n_bt) % (NW * NSLOT) == 0
    idx_t = indices.astype(jnp.int32).T.reshape(-1)
    idx3 = idx_t.reshape(NW, (n_t * n_bt) // NW, K)
    out5 = _make_emb(n_t, n_bt, n_rows, d)(idx3, table)
    return out5.transpose(2, 4, 0, 1, 3).reshape(n_b, n_t, d)
